# Initial kernel scaffold; baseline (speedup 1.0000x reference)
import jax
import jax.numpy as jnp
from jax.experimental import pallas as pl


def kernel(X, Y, X_store, Y_store, pi):
    raise NotImplementedError("write your pallas kernel here")


# speedup vs baseline: 1.2072x; 1.2072x over previous
import functools

import jax
import jax.numpy as jnp
from jax import lax
from jax.experimental import pallas as pl
from jax.experimental.pallas import tpu as pltpu
from jax.experimental.pallas import tpu_sc as plsc

_N = 100000
_DX = 64
_DY = 16
_DK = _DX + _DY
_Q = 256
_BN = 2048
_NBLK = (_N + _BN - 1) // _BN
_BIG = 2**30


def _dist_body(xyT_ref, qn_ref, keysT_ref, nrm_ref, idx_ref, mask_ref,
               bd_ref, bi_ref):
    i = pl.program_id(0)
    ksT = keysT_ref[...]
    kq = lax.dot_general(xyT_ref[...].astype(jnp.bfloat16),
                         ksT.astype(jnp.bfloat16),
                         (((0,), (0,)), ((), ())),
                         preferred_element_type=jnp.float32)
    nrm = nrm_ref[...].reshape(1, _BN)
    dref = (nrm + qn_ref[...]) - 2.0 * kq
    col = lax.broadcasted_iota(jnp.int32, (_Q, _BN), 1)
    valid = (col + i * _BN) < _N
    dref = jnp.where(valid, dref, jnp.inf)
    m = jnp.min(dref, axis=1, keepdims=True)
    bidx = jnp.min(jnp.where(dref == m, col, _BIG), axis=1,
                   keepdims=True) + i * _BN

    @pl.when(i == 0)
    def _():
        bd_ref[...] = m
        bi_ref[...] = bidx

    @pl.when(i > 0)
    def _():
        upd = m < bd_ref[...]
        bd_ref[...] = jnp.where(upd, m, bd_ref[...])
        bi_ref[...] = jnp.where(upd, bidx, bi_ref[...])

    @pl.when(i == _NBLK - 1)
    def _():
        idx_ref[...] = bi_ref[...]
        mask_ref[...] = (bd_ref[...] <= 0.01).astype(jnp.float32)


def _distance_argmin(xyT, qn_col, keysT, norms3):
    return pl.pallas_call(
        _dist_body,
        grid=(_NBLK,),
        in_specs=[
            pl.BlockSpec((_DK, _Q), lambda i: (0, 0)),
            pl.BlockSpec((_Q, 1), lambda i: (0, 0)),
            pl.BlockSpec((_DK, _BN), lambda i: (0, i)),
            pl.BlockSpec((1, 1, _BN), lambda i: (i, 0, 0)),
        ],
        out_specs=[
            pl.BlockSpec((_Q, 1), lambda i: (0, 0)),
            pl.BlockSpec((_Q, 1), lambda i: (0, 0)),
        ],
        out_shape=[
            jax.ShapeDtypeStruct((_Q, 1), jnp.int32),
            jax.ShapeDtypeStruct((_Q, 1), jnp.float32),
        ],
        scratch_shapes=[
            pltpu.VMEM((_Q, 1), jnp.float32),
            pltpu.VMEM((_Q, 1), jnp.int32),
        ],
    )(xyT, qn_col, keysT, norms3)


def _make_sc_gather():
    info = plsc.get_sparse_core_info()
    nc, ns = info.num_cores, info.num_subcores
    nw = nc * ns
    bpw = _Q // nw
    lanes = 16
    mesh = plsc.VectorSubcoreMesh(core_axis_name="c", subcore_axis_name="s")

    @functools.partial(
        pl.kernel,
        mesh=mesh,
        out_type=jax.ShapeDtypeStruct((_Q, _DX), jnp.float32),
        scratch_types=[
            pltpu.VMEM((lanes,), jnp.int32),
            pltpu.VMEM((lanes,), jnp.int32),
            pltpu.VMEM((lanes, 2 * _DX), jnp.float32),
            pltpu.VMEM((bpw, _DX), jnp.float32),
            pltpu.VMEM((bpw, _DX), jnp.float32),
            pltpu.SemaphoreType.DMA,
        ],
    )
    def gather_k(idx_hbm, maskf_hbm, pi_hbm, val2_hbm, out_hbm,
                 idx_v, piv_v, rows_v, mask_v, out_v, sem):
        wid = lax.axis_index("s") * nc + lax.axis_index("c")
        base = wid * bpw
        pltpu.sync_copy(idx_hbm.at[pl.ds(base, bpw)], idx_v.at[pl.ds(0, bpw)])
        pltpu.sync_copy(maskf_hbm.at[pl.ds(base, bpw)], mask_v)
        idx_v[...] = jnp.clip(idx_v[...], 0, _N - 1)
        pltpu.async_copy(pi_hbm.at[idx_v], piv_v, sem).wait()
        pv = piv_v[...]
        half = pv & 1
        piv_v[...] = pv >> 1
        pltpu.async_copy(val2_hbm.at[piv_v], rows_v, sem).wait()
        for q in range(bpw):
            hb = lax.gather(
                half, jnp.full((lanes, 1), q, jnp.int32),
                lax.GatherDimensionNumbers(
                    offset_dims=(), collapsed_slice_dims=(0,),
                    start_index_map=(0,)),
                slice_sizes=(1,),
                mode=lax.GatherScatterMode.PROMISE_IN_BOUNDS)
            hf = hb.astype(jnp.float32)
            for c in range(_DX // lanes):
                lo = rows_v[q, pl.ds(c * lanes, lanes)]
                hi = rows_v[q, pl.ds(_DX + c * lanes, lanes)]
                sel = lo + (hi - lo) * hf
                out_v[q, pl.ds(c * lanes, lanes)] = (
                    sel * mask_v[q, pl.ds(c * lanes, lanes)])
        pltpu.sync_copy(out_v, out_hbm.at[pl.ds(base, bpw)])

    return gather_k


def kernel(X, Y, X_store, Y_store, pi):
    keysT = jnp.concatenate([X_store.T, Y_store.T], axis=0)
    norms = jnp.sum(keysT ** 2, axis=0)
    xyT = jnp.concatenate([X.T, Y.T], axis=0)
    qn_col = jnp.sum(xyT ** 2, axis=0)[:, None]
    norms3 = jnp.pad(norms, (0, _NBLK * _BN - _N)).reshape(_NBLK, 1, _BN)
    idx_col, mask_col = _distance_argmin(xyT, qn_col, keysT, norms3)
    idx = idx_col.reshape(_Q)
    maskf = jnp.broadcast_to(mask_col, (_Q, _DX))
    values2 = X_store.reshape(_N // 2, 2 * _DX)
    gather_fn = _make_sc_gather()
    return gather_fn(idx, maskf, pi.astype(jnp.int32), values2)


# speedup vs baseline: 1.4786x; 1.2248x over previous
import functools

import jax
import jax.numpy as jnp
from jax import lax
from jax.experimental import pallas as pl
from jax.experimental.pallas import tpu as pltpu
from jax.experimental.pallas import tpu_sc as plsc

_N = 100000
_DX = 64
_DY = 16
_DK = _DX + _DY
_Q = 256
_BN = 2048
_NBLK = (_N + _BN - 1) // _BN
_BIG = 2**30


def _norms_reduce(ks2):
    t = ks2[0:8]
    for j in range(1, _DK // 8):
        t = t + ks2[8 * j:8 * (j + 1)]
    t = t[0:4] + t[4:8]
    t = t[0:2] + t[2:4]
    return t[0:1] + t[1:2]


def _dist_body(xyT_ref, qn_ref, xsT_ref, ysT_ref, idx_ref, mask_ref,
               bd_ref, bi_ref):
    i = pl.program_id(0)
    ksT = jnp.concatenate([xsT_ref[...], ysT_ref[...]], axis=0)
    kq = lax.dot_general(xyT_ref[...].astype(jnp.bfloat16),
                         ksT.astype(jnp.bfloat16),
                         (((0,), (0,)), ((), ())),
                         preferred_element_type=jnp.float32)
    nrm = _norms_reduce(ksT * ksT)
    dref = (nrm + qn_ref[...]) - 2.0 * kq
    col = lax.broadcasted_iota(jnp.int32, (_Q, _BN), 1)
    valid = (col + i * _BN) < _N
    dref = jnp.where(valid, dref, jnp.inf)
    m = jnp.min(dref, axis=1, keepdims=True)
    bidx = jnp.min(jnp.where(dref == m, col, _BIG), axis=1,
                   keepdims=True) + i * _BN

    @pl.when(i == 0)
    def _():
        bd_ref[...] = m
        bi_ref[...] = bidx

    @pl.when(i > 0)
    def _():
        upd = m < bd_ref[...]
        bd_ref[...] = jnp.where(upd, m, bd_ref[...])
        bi_ref[...] = jnp.where(upd, bidx, bi_ref[...])

    @pl.when(i == _NBLK - 1)
    def _():
        idx_ref[...] = bi_ref[...]
        mask_ref[...] = (bd_ref[...] <= 0.01).astype(jnp.float32)


def _distance_argmin(xyT, qn_col, xsT, ysT):
    return pl.pallas_call(
        _dist_body,
        grid=(_NBLK,),
        in_specs=[
            pl.BlockSpec((_DK, _Q), lambda i: (0, 0)),
            pl.BlockSpec((_Q, 1), lambda i: (0, 0)),
            pl.BlockSpec((_DX, _BN), lambda i: (0, i)),
            pl.BlockSpec((_DY, _BN), lambda i: (0, i)),
        ],
        out_specs=[
            pl.BlockSpec((_Q, 1), lambda i: (0, 0)),
            pl.BlockSpec((_Q, 1), lambda i: (0, 0)),
        ],
        out_shape=[
            jax.ShapeDtypeStruct((_Q, 1), jnp.int32),
            jax.ShapeDtypeStruct((_Q, 1), jnp.float32),
        ],
        scratch_shapes=[
            pltpu.VMEM((_Q, 1), jnp.float32),
            pltpu.VMEM((_Q, 1), jnp.int32),
        ],
    )(xyT, qn_col, xsT, ysT)


def _make_sc_gather():
    info = plsc.get_sparse_core_info()
    nc, ns = info.num_cores, info.num_subcores
    nw = nc * ns
    bpw = _Q // nw
    lanes = 16
    mesh = plsc.VectorSubcoreMesh(core_axis_name="c", subcore_axis_name="s")

    @functools.partial(
        pl.kernel,
        mesh=mesh,
        out_type=jax.ShapeDtypeStruct((_Q, _DX), jnp.float32),
        scratch_types=[
            pltpu.VMEM((lanes,), jnp.int32),
            pltpu.VMEM((lanes,), jnp.int32),
            pltpu.VMEM((lanes, 2 * _DX), jnp.float32),
            pltpu.VMEM((bpw, _DX), jnp.float32),
            pltpu.VMEM((bpw, _DX), jnp.float32),
            pltpu.SemaphoreType.DMA,
        ],
    )
    def gather_k(idx_hbm, maskf_hbm, pi_hbm, val2_hbm, out_hbm,
                 idx_v, piv_v, rows_v, mask_v, out_v, sem):
        wid = lax.axis_index("s") * nc + lax.axis_index("c")
        base = wid * bpw
        pltpu.sync_copy(idx_hbm.at[pl.ds(base, bpw)], idx_v.at[pl.ds(0, bpw)])
        pltpu.sync_copy(maskf_hbm.at[pl.ds(base, bpw)], mask_v)
        idx_v[...] = jnp.clip(idx_v[...], 0, _N - 1)
        pltpu.async_copy(pi_hbm.at[idx_v], piv_v, sem).wait()
        pv = piv_v[...]
        half = pv & 1
        piv_v[...] = pv >> 1
        pltpu.async_copy(val2_hbm.at[piv_v], rows_v, sem).wait()
        for q in range(bpw):
            hb = lax.gather(
                half, jnp.full((lanes, 1), q, jnp.int32),
                lax.GatherDimensionNumbers(
                    offset_dims=(), collapsed_slice_dims=(0,),
                    start_index_map=(0,)),
                slice_sizes=(1,),
                mode=lax.GatherScatterMode.PROMISE_IN_BOUNDS)
            hf = hb.astype(jnp.float32)
            for c in range(_DX // lanes):
                lo = rows_v[q, pl.ds(c * lanes, lanes)]
                hi = rows_v[q, pl.ds(_DX + c * lanes, lanes)]
                sel = lo + (hi - lo) * hf
                out_v[q, pl.ds(c * lanes, lanes)] = (
                    sel * mask_v[q, pl.ds(c * lanes, lanes)])
        pltpu.sync_copy(out_v, out_hbm.at[pl.ds(base, bpw)])

    return gather_k


def kernel(X, Y, X_store, Y_store, pi):
    xyT = jnp.concatenate([X.T, Y.T], axis=0)
    qn_col = jnp.sum(xyT ** 2, axis=0)[:, None]
    idx_col, mask_col = _distance_argmin(xyT, qn_col, X_store.T, Y_store.T)
    idx = idx_col.reshape(_Q)
    maskf = jnp.broadcast_to(mask_col, (_Q, _DX))
    values2 = X_store.reshape(_N // 2, 2 * _DX)
    gather_fn = _make_sc_gather()
    return gather_fn(idx, maskf, pi.astype(jnp.int32), values2)


# speedup vs baseline: 1.5417x; 1.0427x over previous
import functools

import jax
import jax.numpy as jnp
from jax import lax
from jax.experimental import pallas as pl
from jax.experimental.pallas import tpu as pltpu
from jax.experimental.pallas import tpu_sc as plsc

_N = 100000
_DX = 64
_DY = 16
_DK = _DX + _DY
_Q = 256
_BN = 2048
_NBLK = (_N + _BN - 1) // _BN
_BIG = 2**30


def _norms_reduce(ks2):
    t = ks2[0:8]
    for j in range(1, _DK // 8):
        t = t + ks2[8 * j:8 * (j + 1)]
    t = t[0:4] + t[4:8]
    t = t[0:2] + t[2:4]
    return t[0:1] + t[1:2]


def _dist_body(xyT_ref, qn_ref, xsT_ref, ysT_ref, idx_ref, mask_ref,
               bd_ref, bi_ref):
    i = pl.program_id(0)
    ksT = jnp.concatenate([xsT_ref[...], ysT_ref[...]], axis=0)
    kq = lax.dot_general(xyT_ref[...].astype(jnp.bfloat16),
                         ksT.astype(jnp.bfloat16),
                         (((0,), (0,)), ((), ())),
                         preferred_element_type=jnp.float32)
    nrm = _norms_reduce(ksT * ksT)
    dref = (nrm + qn_ref[...]) - 2.0 * kq
    col = lax.broadcasted_iota(jnp.int32, (_Q, _BN), 1)
    valid = (col + i * _BN) < _N
    dref = jnp.where(valid, dref, jnp.inf)
    m = jnp.min(dref, axis=1, keepdims=True)

    @pl.when(i == 0)
    def _():
        bd_ref[...] = jnp.full((_Q, 1), jnp.inf, jnp.float32)
        bi_ref[...] = jnp.zeros((_Q, 1), jnp.int32)

    better = m < bd_ref[...]

    @pl.when(jnp.any(better))
    def _():
        bidx = jnp.min(jnp.where(dref == m, col, _BIG), axis=1,
                       keepdims=True) + i * _BN
        bd_ref[...] = jnp.where(better, m, bd_ref[...])
        bi_ref[...] = jnp.where(better, bidx, bi_ref[...])

    @pl.when(i == _NBLK - 1)
    def _():
        idx_ref[...] = bi_ref[...]
        mask_ref[...] = (bd_ref[...] <= 0.01).astype(jnp.float32)


def _distance_argmin(xyT, qn_col, xsT, ysT):
    return pl.pallas_call(
        _dist_body,
        grid=(_NBLK,),
        in_specs=[
            pl.BlockSpec((_DK, _Q), lambda i: (0, 0)),
            pl.BlockSpec((_Q, 1), lambda i: (0, 0)),
            pl.BlockSpec((_DX, _BN), lambda i: (0, i)),
            pl.BlockSpec((_DY, _BN), lambda i: (0, i)),
        ],
        out_specs=[
            pl.BlockSpec((_Q, 1), lambda i: (0, 0)),
            pl.BlockSpec((_Q, 1), lambda i: (0, 0)),
        ],
        out_shape=[
            jax.ShapeDtypeStruct((_Q, 1), jnp.int32),
            jax.ShapeDtypeStruct((_Q, 1), jnp.float32),
        ],
        scratch_shapes=[
            pltpu.VMEM((_Q, 1), jnp.float32),
            pltpu.VMEM((_Q, 1), jnp.int32),
        ],
    )(xyT, qn_col, xsT, ysT)


def _make_sc_gather():
    info = plsc.get_sparse_core_info()
    nc, ns = info.num_cores, info.num_subcores
    nw = nc * ns
    bpw = _Q // nw
    lanes = 16
    mesh = plsc.VectorSubcoreMesh(core_axis_name="c", subcore_axis_name="s")

    @functools.partial(
        pl.kernel,
        mesh=mesh,
        out_type=jax.ShapeDtypeStruct((_Q, _DX), jnp.float32),
        scratch_types=[
            pltpu.VMEM((lanes,), jnp.int32),
            pltpu.VMEM((lanes,), jnp.int32),
            pltpu.VMEM((lanes, 2 * _DX), jnp.float32),
            pltpu.VMEM((bpw, _DX), jnp.float32),
            pltpu.VMEM((bpw, _DX), jnp.float32),
            pltpu.SemaphoreType.DMA,
        ],
    )
    def gather_k(idx_hbm, maskf_hbm, pi_hbm, val2_hbm, out_hbm,
                 idx_v, piv_v, rows_v, mask_v, out_v, sem):
        wid = lax.axis_index("s") * nc + lax.axis_index("c")
        base = wid * bpw
        pltpu.sync_copy(idx_hbm.at[pl.ds(base, bpw)], idx_v.at[pl.ds(0, bpw)])
        pltpu.sync_copy(maskf_hbm.at[pl.ds(base, bpw)], mask_v)
        idx_v[...] = jnp.clip(idx_v[...], 0, _N - 1)
        pltpu.async_copy(pi_hbm.at[idx_v], piv_v, sem).wait()
        pv = piv_v[...]
        half = pv & 1
        piv_v[...] = pv >> 1
        pltpu.async_copy(val2_hbm.at[piv_v], rows_v, sem).wait()
        for q in range(bpw):
            hb = lax.gather(
                half, jnp.full((lanes, 1), q, jnp.int32),
                lax.GatherDimensionNumbers(
                    offset_dims=(), collapsed_slice_dims=(0,),
                    start_index_map=(0,)),
                slice_sizes=(1,),
                mode=lax.GatherScatterMode.PROMISE_IN_BOUNDS)
            hf = hb.astype(jnp.float32)
            for c in range(_DX // lanes):
                lo = rows_v[q, pl.ds(c * lanes, lanes)]
                hi = rows_v[q, pl.ds(_DX + c * lanes, lanes)]
                sel = lo + (hi - lo) * hf
                out_v[q, pl.ds(c * lanes, lanes)] = (
                    sel * mask_v[q, pl.ds(c * lanes, lanes)])
        pltpu.sync_copy(out_v, out_hbm.at[pl.ds(base, bpw)])

    return gather_k


def kernel(X, Y, X_store, Y_store, pi):
    xyT = jnp.concatenate([X.T, Y.T], axis=0)
    qn_col = jnp.sum(xyT ** 2, axis=0)[:, None]
    idx_col, mask_col = _distance_argmin(xyT, qn_col, X_store.T, Y_store.T)
    idx = idx_col.reshape(_Q)
    maskf = jnp.broadcast_to(mask_col, (_Q, _DX))
    values2 = X_store.reshape(_N // 2, 2 * _DX)
    gather_fn = _make_sc_gather()
    return gather_fn(idx, maskf, pi.astype(jnp.int32), values2)


# speedup vs baseline: 2.0792x; 1.3486x over previous
import functools

import jax
import jax.numpy as jnp
from jax import lax
from jax.experimental import pallas as pl
from jax.experimental.pallas import tpu as pltpu
from jax.experimental.pallas import tpu_sc as plsc

_N = 100000
_DX = 64
_DY = 16
_DK = _DX + _DY
_Q = 256
_BN = 2048
_NBLK = (_N + _BN - 1) // _BN
_BIG = 2**30


def _norms_reduce(ks2):
    t = ks2[0:8]
    for j in range(1, _DK // 8):
        t = t + ks2[8 * j:8 * (j + 1)]
    t = t[0:4] + t[4:8]
    t = t[0:2] + t[2:4]
    return t[0:1] + t[1:2]


def _dist_body(xyT_ref, qn_ref, xsT_ref, ysT_ref, idx_ref, mask_ref,
               val2_ref, bd_ref, bi_ref):
    i = pl.program_id(0)
    xsT = xsT_ref[...]
    t = jnp.transpose(xsT)
    val2_ref[...] = jnp.concatenate(
        [t[0:_BN // 2], t[_BN // 2:_BN]], axis=1)
    ksT = jnp.concatenate([xsT, ysT_ref[...]], axis=0)
    kq = lax.dot_general(xyT_ref[...].astype(jnp.bfloat16),
                         ksT.astype(jnp.bfloat16),
                         (((0,), (0,)), ((), ())),
                         preferred_element_type=jnp.float32)
    nrm = _norms_reduce(ksT * ksT)
    dref = (nrm + qn_ref[...]) - 2.0 * kq
    col = lax.broadcasted_iota(jnp.int32, (_Q, _BN), 1)
    valid = (col + i * _BN) < _N
    dref = jnp.where(valid, dref, jnp.inf)
    m = jnp.min(dref, axis=1, keepdims=True)

    @pl.when(i == 0)
    def _():
        bd_ref[...] = jnp.full((_Q, 1), jnp.inf, jnp.float32)
        bi_ref[...] = jnp.zeros((_Q, 1), jnp.int32)

    better = m < bd_ref[...]

    @pl.when(jnp.any(better))
    def _():
        bidx = jnp.min(jnp.where(dref == m, col, _BIG), axis=1,
                       keepdims=True) + i * _BN
        bd_ref[...] = jnp.where(better, m, bd_ref[...])
        bi_ref[...] = jnp.where(better, bidx, bi_ref[...])

    @pl.when(i == _NBLK - 1)
    def _():
        idx_ref[...] = bi_ref[...]
        mask_ref[...] = (bd_ref[...] <= 0.01).astype(jnp.float32)


def _distance_argmin(xyT, qn_col, xsT, ysT):
    return pl.pallas_call(
        _dist_body,
        grid=(_NBLK,),
        in_specs=[
            pl.BlockSpec((_DK, _Q), lambda i: (0, 0)),
            pl.BlockSpec((_Q, 1), lambda i: (0, 0)),
            pl.BlockSpec((_DX, _BN), lambda i: (0, i)),
            pl.BlockSpec((_DY, _BN), lambda i: (0, i)),
        ],
        out_specs=[
            pl.BlockSpec((_Q, 1), lambda i: (0, 0)),
            pl.BlockSpec((_Q, 1), lambda i: (0, 0)),
            pl.BlockSpec((_BN // 2, 2 * _DX), lambda i: (i, 0)),
        ],
        out_shape=[
            jax.ShapeDtypeStruct((_Q, 1), jnp.int32),
            jax.ShapeDtypeStruct((_Q, 1), jnp.float32),
            jax.ShapeDtypeStruct((_NBLK * _BN // 2, 2 * _DX), jnp.float32),
        ],
        scratch_shapes=[
            pltpu.VMEM((_Q, 1), jnp.float32),
            pltpu.VMEM((_Q, 1), jnp.int32),
        ],
    )(xyT, qn_col, xsT, ysT)


def _make_sc_gather():
    info = plsc.get_sparse_core_info()
    nc, ns = info.num_cores, info.num_subcores
    nw = nc * ns
    bpw = _Q // nw
    lanes = 16
    mesh = plsc.VectorSubcoreMesh(core_axis_name="c", subcore_axis_name="s")

    @functools.partial(
        pl.kernel,
        mesh=mesh,
        out_type=jax.ShapeDtypeStruct((_Q, _DX), jnp.float32),
        scratch_types=[
            pltpu.VMEM((lanes,), jnp.int32),
            pltpu.VMEM((lanes,), jnp.int32),
            pltpu.VMEM((lanes, 2 * _DX), jnp.float32),
            pltpu.VMEM((bpw, _DX), jnp.float32),
            pltpu.VMEM((bpw, _DX), jnp.float32),
            pltpu.SemaphoreType.DMA,
        ],
    )
    def gather_k(idx_hbm, maskf_hbm, pi_hbm, val2_hbm, out_hbm,
                 idx_v, piv_v, rows_v, mask_v, out_v, sem):
        wid = lax.axis_index("s") * nc + lax.axis_index("c")
        base = wid * bpw
        pltpu.sync_copy(idx_hbm.at[pl.ds(base, bpw)], idx_v.at[pl.ds(0, bpw)])
        pltpu.sync_copy(maskf_hbm.at[pl.ds(base, bpw)], mask_v)
        idx_v[...] = jnp.clip(idx_v[...], 0, _N - 1)
        pltpu.async_copy(pi_hbm.at[idx_v], piv_v, sem).wait()
        pv = piv_v[...]
        half = (pv >> 10) & 1
        piv_v[...] = ((pv >> 11) << 10) | (pv & 1023)
        pltpu.async_copy(val2_hbm.at[piv_v], rows_v, sem).wait()
        for q in range(bpw):
            hb = lax.gather(
                half, jnp.full((lanes, 1), q, jnp.int32),
                lax.GatherDimensionNumbers(
                    offset_dims=(), collapsed_slice_dims=(0,),
                    start_index_map=(0,)),
                slice_sizes=(1,),
                mode=lax.GatherScatterMode.PROMISE_IN_BOUNDS)
            hf = hb.astype(jnp.float32)
            for c in range(_DX // lanes):
                lo = rows_v[q, pl.ds(c * lanes, lanes)]
                hi = rows_v[q, pl.ds(_DX + c * lanes, lanes)]
                sel = lo + (hi - lo) * hf
                out_v[q, pl.ds(c * lanes, lanes)] = (
                    sel * mask_v[q, pl.ds(c * lanes, lanes)])
        pltpu.sync_copy(out_v, out_hbm.at[pl.ds(base, bpw)])

    return gather_k


def kernel(X, Y, X_store, Y_store, pi):
    xyT = jnp.concatenate([X.T, Y.T], axis=0)
    qn_col = jnp.sum(xyT ** 2, axis=0)[:, None]
    idx_col, mask_col, values2 = _distance_argmin(
        xyT, qn_col, X_store.T, Y_store.T)
    idx = idx_col.reshape(_Q)
    maskf = jnp.broadcast_to(mask_col, (_Q, _DX))
    gather_fn = _make_sc_gather()
    return gather_fn(idx, maskf, pi.astype(jnp.int32), values2)


# speedup vs baseline: 2.3471x; 1.1288x over previous
import functools

import jax
import jax.numpy as jnp
from jax import lax
from jax.experimental import pallas as pl
from jax.experimental.pallas import tpu as pltpu
from jax.experimental.pallas import tpu_sc as plsc

_N = 100000
_DX = 64
_DY = 16
_DK = _DX + _DY
_Q = 256
_BN = 4096
_LOGBN = _BN.bit_length() - 1
_NBLK = (_N + _BN - 1) // _BN
_BIG = 2**30


def _norms_reduce(ks2):
    t = ks2[0:8]
    for j in range(1, _DK // 8):
        t = t + ks2[8 * j:8 * (j + 1)]
    t = t[0:4] + t[4:8]
    t = t[0:2] + t[2:4]
    return t[0:1] + t[1:2]


def _dist_body(xyT_ref, qn_ref, xsT_ref, ysT_ref, idx_ref, mask_ref,
               val2_ref, bd_ref, bi_ref):
    i = pl.program_id(0)
    xsT = xsT_ref[...]
    t = jnp.transpose(xsT)
    val2_ref[...] = jnp.concatenate(
        [t[0:_BN // 2], t[_BN // 2:_BN]], axis=1)
    ksT = jnp.concatenate([xsT, ysT_ref[...]], axis=0)
    lane = lax.broadcasted_iota(jnp.int32, (1, _BN), 1)
    valid = (lane + i * _BN) < _N
    ksT = jnp.where(valid, ksT, 0.0)
    kq = lax.dot_general(xyT_ref[...].astype(jnp.bfloat16),
                         ksT.astype(jnp.bfloat16),
                         (((0,), (0,)), ((), ())),
                         preferred_element_type=jnp.float32)
    nrm = jnp.where(valid, _norms_reduce(ksT * ksT), jnp.inf)
    dref = (nrm + qn_ref[...]) - 2.0 * kq
    m = jnp.min(dref, axis=1, keepdims=True)

    @pl.when(i == 0)
    def _():
        bd_ref[...] = jnp.full((_Q, 1), jnp.inf, jnp.float32)
        bi_ref[...] = jnp.zeros((_Q, 1), jnp.int32)

    better = m < bd_ref[...]

    @pl.when(jnp.any(better))
    def _():
        col = lax.broadcasted_iota(jnp.int32, (_Q, _BN), 1)
        bidx = jnp.min(jnp.where(dref == m, col, _BIG), axis=1,
                       keepdims=True) + i * _BN
        bd_ref[...] = jnp.where(better, m, bd_ref[...])
        bi_ref[...] = jnp.where(better, bidx, bi_ref[...])

    @pl.when(i == _NBLK - 1)
    def _():
        idx_ref[...] = bi_ref[...]
        mask_ref[...] = (bd_ref[...] <= 0.01).astype(jnp.float32)


def _distance_argmin(xyT, qn_col, xsT, ysT):
    return pl.pallas_call(
        _dist_body,
        grid=(_NBLK,),
        in_specs=[
            pl.BlockSpec((_DK, _Q), lambda i: (0, 0)),
            pl.BlockSpec((_Q, 1), lambda i: (0, 0)),
            pl.BlockSpec((_DX, _BN), lambda i: (0, i)),
            pl.BlockSpec((_DY, _BN), lambda i: (0, i)),
        ],
        out_specs=[
            pl.BlockSpec((_Q, 1), lambda i: (0, 0)),
            pl.BlockSpec((_Q, 1), lambda i: (0, 0)),
            pl.BlockSpec((_BN // 2, 2 * _DX), lambda i: (i, 0)),
        ],
        out_shape=[
            jax.ShapeDtypeStruct((_Q, 1), jnp.int32),
            jax.ShapeDtypeStruct((_Q, 1), jnp.float32),
            jax.ShapeDtypeStruct((_NBLK * _BN // 2, 2 * _DX), jnp.float32),
        ],
        scratch_shapes=[
            pltpu.VMEM((_Q, 1), jnp.float32),
            pltpu.VMEM((_Q, 1), jnp.int32),
        ],
    )(xyT, qn_col, xsT, ysT)


def _make_sc_gather():
    info = plsc.get_sparse_core_info()
    nc, ns = info.num_cores, info.num_subcores
    nw = nc * ns
    bpw = _Q // nw
    lanes = 16
    mesh = plsc.VectorSubcoreMesh(core_axis_name="c", subcore_axis_name="s")

    @functools.partial(
        pl.kernel,
        mesh=mesh,
        out_type=jax.ShapeDtypeStruct((_Q, _DX), jnp.float32),
        scratch_types=[
            pltpu.VMEM((lanes,), jnp.int32),
            pltpu.VMEM((lanes,), jnp.int32),
            pltpu.VMEM((lanes, 2 * _DX), jnp.float32),
            pltpu.VMEM((bpw, _DX), jnp.float32),
            pltpu.VMEM((bpw, _DX), jnp.float32),
            pltpu.SemaphoreType.DMA,
        ],
    )
    def gather_k(idx_hbm, maskf_hbm, pi_hbm, val2_hbm, out_hbm,
                 idx_v, piv_v, rows_v, mask_v, out_v, sem):
        wid = lax.axis_index("s") * nc + lax.axis_index("c")
        base = wid * bpw
        pltpu.sync_copy(idx_hbm.at[pl.ds(base, bpw)], idx_v.at[pl.ds(0, bpw)])
        pltpu.sync_copy(maskf_hbm.at[pl.ds(base, bpw)], mask_v)
        idx_v[...] = jnp.clip(idx_v[...], 0, _N - 1)
        pltpu.async_copy(pi_hbm.at[idx_v], piv_v, sem).wait()
        pv = piv_v[...]
        half = (pv >> (_LOGBN - 1)) & 1
        piv_v[...] = ((pv >> _LOGBN) << (_LOGBN - 1)) | (pv & (_BN // 2 - 1))
        pltpu.async_copy(val2_hbm.at[piv_v], rows_v, sem).wait()
        for q in range(bpw):
            hb = lax.gather(
                half, jnp.full((lanes, 1), q, jnp.int32),
                lax.GatherDimensionNumbers(
                    offset_dims=(), collapsed_slice_dims=(0,),
                    start_index_map=(0,)),
                slice_sizes=(1,),
                mode=lax.GatherScatterMode.PROMISE_IN_BOUNDS)
            hf = hb.astype(jnp.float32)
            for c in range(_DX // lanes):
                lo = rows_v[q, pl.ds(c * lanes, lanes)]
                hi = rows_v[q, pl.ds(_DX + c * lanes, lanes)]
                sel = lo + (hi - lo) * hf
                out_v[q, pl.ds(c * lanes, lanes)] = (
                    sel * mask_v[q, pl.ds(c * lanes, lanes)])
        pltpu.sync_copy(out_v, out_hbm.at[pl.ds(base, bpw)])

    return gather_k


def kernel(X, Y, X_store, Y_store, pi):
    xyT = jnp.concatenate([X.T, Y.T], axis=0)
    qn_col = jnp.sum(xyT ** 2, axis=0)[:, None]
    idx_col, mask_col, values2 = _distance_argmin(
        xyT, qn_col, X_store.T, Y_store.T)
    idx = idx_col.reshape(_Q)
    maskf = jnp.broadcast_to(mask_col, (_Q, _DX))
    gather_fn = _make_sc_gather()
    return gather_fn(idx, maskf, pi.astype(jnp.int32), values2)


# speedup vs baseline: 2.4120x; 1.0277x over previous
import functools

import jax
import jax.numpy as jnp
from jax import lax
from jax.experimental import pallas as pl
from jax.experimental.pallas import tpu as pltpu
from jax.experimental.pallas import tpu_sc as plsc

_N = 100000
_DX = 64
_DY = 16
_DK = _DX + _DY
_Q = 256
_BN = 8192
_LOGBN = _BN.bit_length() - 1
_NBLK = (_N + _BN - 1) // _BN
_BIG = 2**30


def _norms_reduce(ks2):
    t = ks2[0:8]
    for j in range(1, _DK // 8):
        t = t + ks2[8 * j:8 * (j + 1)]
    t = t[0:4] + t[4:8]
    t = t[0:2] + t[2:4]
    return t[0:1] + t[1:2]


def _dist_body(xyT_ref, qn_ref, xsT_ref, ysT_ref, idx_ref, mask_ref,
               val2_ref, bd_ref, bi_ref):
    i = pl.program_id(0)
    xsT = xsT_ref[...]
    t = jnp.transpose(xsT)
    val2_ref[...] = jnp.concatenate(
        [t[0:_BN // 2], t[_BN // 2:_BN]], axis=1)
    ksT = jnp.concatenate([xsT, ysT_ref[...]], axis=0)
    lane = lax.broadcasted_iota(jnp.int32, (1, _BN), 1)
    valid = (lane + i * _BN) < _N
    ksT = jnp.where(valid, ksT, 0.0)
    kq = lax.dot_general(xyT_ref[...].astype(jnp.bfloat16),
                         ksT.astype(jnp.bfloat16),
                         (((0,), (0,)), ((), ())),
                         preferred_element_type=jnp.float32)
    nrm = jnp.where(valid, _norms_reduce(ksT * ksT), jnp.inf)
    dref = (nrm + qn_ref[...]) - 2.0 * kq
    m = jnp.min(dref, axis=1, keepdims=True)

    @pl.when(i == 0)
    def _():
        bd_ref[...] = jnp.full((_Q, 1), jnp.inf, jnp.float32)
        bi_ref[...] = jnp.zeros((_Q, 1), jnp.int32)

    better = m < bd_ref[...]

    @pl.when(jnp.any(better))
    def _():
        col = lax.broadcasted_iota(jnp.int32, (_Q, _BN), 1)
        bidx = jnp.min(jnp.where(dref == m, col, _BIG), axis=1,
                       keepdims=True) + i * _BN
        bd_ref[...] = jnp.where(better, m, bd_ref[...])
        bi_ref[...] = jnp.where(better, bidx, bi_ref[...])

    @pl.when(i == _NBLK - 1)
    def _():
        idx_ref[...] = bi_ref[...]
        mask_ref[...] = (bd_ref[...] <= 0.01).astype(jnp.float32)


def _distance_argmin(xyT, qn_col, xsT, ysT):
    return pl.pallas_call(
        _dist_body,
        grid=(_NBLK,),
        in_specs=[
            pl.BlockSpec((_DK, _Q), lambda i: (0, 0)),
            pl.BlockSpec((_Q, 1), lambda i: (0, 0)),
            pl.BlockSpec((_DX, _BN), lambda i: (0, i)),
            pl.BlockSpec((_DY, _BN), lambda i: (0, i)),
        ],
        out_specs=[
            pl.BlockSpec((_Q, 1), lambda i: (0, 0)),
            pl.BlockSpec((_Q, 1), lambda i: (0, 0)),
            pl.BlockSpec((_BN // 2, 2 * _DX), lambda i: (i, 0)),
        ],
        out_shape=[
            jax.ShapeDtypeStruct((_Q, 1), jnp.int32),
            jax.ShapeDtypeStruct((_Q, 1), jnp.float32),
            jax.ShapeDtypeStruct((_NBLK * _BN // 2, 2 * _DX), jnp.float32),
        ],
        scratch_shapes=[
            pltpu.VMEM((_Q, 1), jnp.float32),
            pltpu.VMEM((_Q, 1), jnp.int32),
        ],
    )(xyT, qn_col, xsT, ysT)


def _make_sc_gather():
    info = plsc.get_sparse_core_info()
    nc, ns = info.num_cores, info.num_subcores
    nw = nc * ns
    bpw = _Q // nw
    lanes = 16
    mesh = plsc.VectorSubcoreMesh(core_axis_name="c", subcore_axis_name="s")

    @functools.partial(
        pl.kernel,
        mesh=mesh,
        out_type=jax.ShapeDtypeStruct((_Q, _DX), jnp.float32),
        scratch_types=[
            pltpu.VMEM((lanes,), jnp.int32),
            pltpu.VMEM((lanes,), jnp.int32),
            pltpu.VMEM((lanes, 2 * _DX), jnp.float32),
            pltpu.VMEM((bpw, _DX), jnp.float32),
            pltpu.VMEM((bpw, _DX), jnp.float32),
            pltpu.SemaphoreType.DMA,
        ],
    )
    def gather_k(idx_hbm, maskf_hbm, pi_hbm, val2_hbm, out_hbm,
                 idx_v, piv_v, rows_v, mask_v, out_v, sem):
        wid = lax.axis_index("s") * nc + lax.axis_index("c")
        base = wid * bpw
        pltpu.sync_copy(idx_hbm.at[pl.ds(base, bpw)], idx_v.at[pl.ds(0, bpw)])
        pltpu.sync_copy(maskf_hbm.at[pl.ds(base, bpw)], mask_v)
        idx_v[...] = jnp.clip(idx_v[...], 0, _N - 1)
        pltpu.async_copy(pi_hbm.at[idx_v], piv_v, sem).wait()
        pv = piv_v[...]
        half = (pv >> (_LOGBN - 1)) & 1
        piv_v[...] = ((pv >> _LOGBN) << (_LOGBN - 1)) | (pv & (_BN // 2 - 1))
        pltpu.async_copy(val2_hbm.at[piv_v], rows_v, sem).wait()
        for q in range(bpw):
            hb = lax.gather(
                half, jnp.full((lanes, 1), q, jnp.int32),
                lax.GatherDimensionNumbers(
                    offset_dims=(), collapsed_slice_dims=(0,),
                    start_index_map=(0,)),
                slice_sizes=(1,),
                mode=lax.GatherScatterMode.PROMISE_IN_BOUNDS)
            hf = hb.astype(jnp.float32)
            for c in range(_DX // lanes):
                lo = rows_v[q, pl.ds(c * lanes, lanes)]
                hi = rows_v[q, pl.ds(_DX + c * lanes, lanes)]
                sel = lo + (hi - lo) * hf
                out_v[q, pl.ds(c * lanes, lanes)] = (
                    sel * mask_v[q, pl.ds(c * lanes, lanes)])
        pltpu.sync_copy(out_v, out_hbm.at[pl.ds(base, bpw)])

    return gather_k


def kernel(X, Y, X_store, Y_store, pi):
    xyT = jnp.concatenate([X.T, Y.T], axis=0)
    qn_col = jnp.sum(xyT ** 2, axis=0)[:, None]
    idx_col, mask_col, values2 = _distance_argmin(
        xyT, qn_col, X_store.T, Y_store.T)
    idx = idx_col.reshape(_Q)
    maskf = jnp.broadcast_to(mask_col, (_Q, _DX))
    gather_fn = _make_sc_gather()
    return gather_fn(idx, maskf, pi.astype(jnp.int32), values2)


# speedup vs baseline: 2.4171x; 1.0021x over previous
import functools

import jax
import jax.numpy as jnp
from jax import lax
from jax.experimental import pallas as pl
from jax.experimental.pallas import tpu as pltpu
from jax.experimental.pallas import tpu_sc as plsc

_N = 100000
_DX = 64
_DY = 16
_DK = _DX + _DY
_Q = 256
_BN = 8192
_LOGBN = _BN.bit_length() - 1
_NBLK = (_N + _BN - 1) // _BN
_BIG = 2**30


def _norms_reduce(ks2):
    t = ks2[0:8]
    for j in range(1, _DK // 8):
        t = t + ks2[8 * j:8 * (j + 1)]
    t = t[0:4] + t[4:8]
    t = t[0:2] + t[2:4]
    return t[0:1] + t[1:2]


def _dist_body(xyT_ref, qn_ref, xsT_ref, ysT_ref, idx_ref, mask_ref,
               val2_ref, bd_ref, bi_ref):
    i = pl.program_id(0)
    xsT = xsT_ref[...]
    t = jnp.transpose(xsT)
    val2_ref[...] = jnp.concatenate(
        [t[0:_BN // 2], t[_BN // 2:_BN]], axis=1)
    ksT = jnp.concatenate([xsT, ysT_ref[...]], axis=0)
    lane = lax.broadcasted_iota(jnp.int32, (1, _BN), 1)
    valid = (lane + i * _BN) < _N
    ksT = jnp.where(valid, ksT, 0.0)
    kq = lax.dot_general(xyT_ref[...].astype(jnp.bfloat16),
                         ksT.astype(jnp.bfloat16),
                         (((0,), (0,)), ((), ())),
                         preferred_element_type=jnp.float32)
    nrm = jnp.where(valid, _norms_reduce(ksT * ksT), jnp.inf)
    m = jnp.min((nrm + qn_ref[...]) - 2.0 * kq, axis=1,
                keepdims=True)

    @pl.when(i == 0)
    def _():
        bd_ref[...] = jnp.full((_Q, 1), jnp.inf, jnp.float32)
        bi_ref[...] = jnp.zeros((_Q, 1), jnp.int32)

    better = m < bd_ref[...]

    @pl.when(jnp.any(better))
    def _():
        dref = (nrm + qn_ref[...]) - 2.0 * kq
        col = lax.broadcasted_iota(jnp.int32, (_Q, _BN), 1)
        bidx = jnp.min(jnp.where(dref == m, col, _BIG), axis=1,
                       keepdims=True) + i * _BN
        bd_ref[...] = jnp.where(better, m, bd_ref[...])
        bi_ref[...] = jnp.where(better, bidx, bi_ref[...])

    @pl.when(i == _NBLK - 1)
    def _():
        idx_ref[...] = bi_ref[...]
        mask_ref[...] = (bd_ref[...] <= 0.01).astype(jnp.float32)


def _distance_argmin(xyT, qn_col, xsT, ysT):
    return pl.pallas_call(
        _dist_body,
        grid=(_NBLK,),
        in_specs=[
            pl.BlockSpec((_DK, _Q), lambda i: (0, 0)),
            pl.BlockSpec((_Q, 1), lambda i: (0, 0)),
            pl.BlockSpec((_DX, _BN), lambda i: (0, i)),
            pl.BlockSpec((_DY, _BN), lambda i: (0, i)),
        ],
        out_specs=[
            pl.BlockSpec((_Q, 1), lambda i: (0, 0)),
            pl.BlockSpec((_Q, 1), lambda i: (0, 0)),
            pl.BlockSpec((_BN // 2, 2 * _DX), lambda i: (i, 0)),
        ],
        out_shape=[
            jax.ShapeDtypeStruct((_Q, 1), jnp.int32),
            jax.ShapeDtypeStruct((_Q, 1), jnp.float32),
            jax.ShapeDtypeStruct((_NBLK * _BN // 2, 2 * _DX), jnp.float32),
        ],
        scratch_shapes=[
            pltpu.VMEM((_Q, 1), jnp.float32),
            pltpu.VMEM((_Q, 1), jnp.int32),
        ],
    )(xyT, qn_col, xsT, ysT)


def _make_sc_gather():
    info = plsc.get_sparse_core_info()
    nc, ns = info.num_cores, info.num_subcores
    nw = nc * ns
    bpw = _Q // nw
    lanes = 16
    mesh = plsc.VectorSubcoreMesh(core_axis_name="c", subcore_axis_name="s")

    @functools.partial(
        pl.kernel,
        mesh=mesh,
        out_type=jax.ShapeDtypeStruct((_Q, _DX), jnp.float32),
        scratch_types=[
            pltpu.VMEM((lanes,), jnp.int32),
            pltpu.VMEM((lanes,), jnp.int32),
            pltpu.VMEM((lanes, 2 * _DX), jnp.float32),
            pltpu.VMEM((bpw, _DX), jnp.float32),
            pltpu.VMEM((bpw, _DX), jnp.float32),
            pltpu.SemaphoreType.DMA,
        ],
    )
    def gather_k(idx_hbm, maskf_hbm, pi_hbm, val2_hbm, out_hbm,
                 idx_v, piv_v, rows_v, mask_v, out_v, sem):
        wid = lax.axis_index("s") * nc + lax.axis_index("c")
        base = wid * bpw
        pltpu.sync_copy(idx_hbm.at[pl.ds(base, bpw)], idx_v.at[pl.ds(0, bpw)])
        pltpu.sync_copy(maskf_hbm.at[pl.ds(base, bpw)], mask_v)
        idx_v[...] = jnp.clip(idx_v[...], 0, _N - 1)
        pltpu.async_copy(pi_hbm.at[idx_v], piv_v, sem).wait()
        pv = piv_v[...]
        half = (pv >> (_LOGBN - 1)) & 1
        piv_v[...] = ((pv >> _LOGBN) << (_LOGBN - 1)) | (pv & (_BN // 2 - 1))
        pltpu.async_copy(val2_hbm.at[piv_v], rows_v, sem).wait()
        for q in range(bpw):
            hb = lax.gather(
                half, jnp.full((lanes, 1), q, jnp.int32),
                lax.GatherDimensionNumbers(
                    offset_dims=(), collapsed_slice_dims=(0,),
                    start_index_map=(0,)),
                slice_sizes=(1,),
                mode=lax.GatherScatterMode.PROMISE_IN_BOUNDS)
            hf = hb.astype(jnp.float32)
            for c in range(_DX // lanes):
                lo = rows_v[q, pl.ds(c * lanes, lanes)]
                hi = rows_v[q, pl.ds(_DX + c * lanes, lanes)]
                sel = lo + (hi - lo) * hf
                out_v[q, pl.ds(c * lanes, lanes)] = (
                    sel * mask_v[q, pl.ds(c * lanes, lanes)])
        pltpu.sync_copy(out_v, out_hbm.at[pl.ds(base, bpw)])

    return gather_k


def kernel(X, Y, X_store, Y_store, pi):
    xyT = jnp.concatenate([X.T, Y.T], axis=0)
    qn_col = jnp.sum(xyT ** 2, axis=0)[:, None]
    idx_col, mask_col, values2 = _distance_argmin(
        xyT, qn_col, X_store.T, Y_store.T)
    idx = idx_col.reshape(_Q)
    maskf = jnp.broadcast_to(mask_col, (_Q, _DX))
    gather_fn = _make_sc_gather()
    return gather_fn(idx, maskf, pi.astype(jnp.int32), values2)
